# Initial kernel scaffold; baseline (speedup 1.0000x reference)
import jax
import jax.numpy as jnp
from jax.experimental import pallas as pl


def kernel(Z, R, emb, params):
    raise NotImplementedError("write your pallas kernel here")


# speedup vs baseline: 8.8724x; 8.8724x over previous
import math

import jax
import jax.numpy as jnp
from jax.experimental import pallas as pl
from jax.experimental.pallas import tpu as pltpu

B = 8
N = 256
D = 128
N_RBF = 20
RBF_PAD = 32
CUTOFF = 5.0
TI = 64
TJ = 64
NT = N // TI
LN2 = math.log(2.0)

_width = CUTOFF / (N_RBF - 1)
_COEFF = -0.5 / (_width * _width)


def _ssp(v):
    return jax.nn.softplus(v) - LN2


def _layer_body(xi_ref, xj_ref, ri_ref, rj_ref, w_in_ref, b_in_ref,
                w_f1_ref, b_f1_ref, w_f2_ref, b_f2_ref,
                w_o1_ref, b_o1_ref, w_o2_ref, b_o2_ref,
                y_ref, acc_ref):
    i = pl.program_id(1)
    j = pl.program_id(2)

    @pl.when(j >= i)
    def _compute():
        ri = ri_ref[0, 0]
        rj = rj_ref[0, 0]
        dims = (((1,), (1,)), ((), ()))
        prod = jax.lax.dot_general(ri, rj, dims,
                                   preferred_element_type=jnp.float32)
        ri2 = jnp.sum(ri * ri, axis=1, keepdims=True)
        rj2 = jax.lax.dot_general(jnp.ones((1, 8), jnp.float32), rj * rj, dims,
                                  preferred_element_type=jnp.float32)
        d2 = ri2 + rj2 - 2.0 * prod
        d = jnp.sqrt(jnp.maximum(d2, 0.0) + 1e-12)

        ii = i * TI + jax.lax.broadcasted_iota(jnp.int32, (TI, TJ), 0)
        jj = j * TJ + jax.lax.broadcasted_iota(jnp.int32, (TI, TJ), 1)
        mask = (jj > ii) & (d < CUTOFF)
        rcut = jnp.where(mask, 0.5 * (jnp.cos(d * (math.pi / CUTOFF)) + 1.0), 0.0)

        k = jax.lax.broadcasted_iota(jnp.int32, (1, 1, RBF_PAD), 2)
        offs = jnp.where(k < N_RBF, k.astype(jnp.float32) * _width, 1e6)
        delta = d[:, :, None] - offs
        f = jnp.exp(_COEFF * (delta * delta)).reshape(TI * TJ, RBF_PAD)

        t1 = jnp.dot(f, w_f1_ref[...], preferred_element_type=jnp.float32)
        t1 = _ssp(t1 + b_f1_ref[...])
        w = jnp.dot(t1, w_f2_ref[...], preferred_element_type=jnp.float32)
        w3 = (w + b_f2_ref[...]).reshape(TI, TJ, D) * rcut[:, :, None]

        h_j = jnp.dot(xj_ref[0], w_in_ref[...],
                      preferred_element_type=jnp.float32) + b_in_ref[...]
        msg = jnp.sum(w3 * h_j[None, :, :], axis=1)

        @pl.when(j == i)
        def _init():
            acc_ref[...] = msg

        @pl.when(j > i)
        def _add():
            acc_ref[...] = acc_ref[...] + msg

    @pl.when(j == NT - 1)
    def _finalize():
        agg = acc_ref[...]
        o = _ssp(jnp.dot(agg, w_o1_ref[...], preferred_element_type=jnp.float32)
                 + b_o1_ref[...])
        out = jnp.dot(o, w_o2_ref[...], preferred_element_type=jnp.float32) \
            + b_o2_ref[...]
        y_ref[0, :, :] = xi_ref[0] + out


def _interaction_layer(x, rp, p, wf1p):
    wspec = pl.BlockSpec((D, D), lambda b, i, j: (0, 0))
    bspec = pl.BlockSpec((1, D), lambda b, i, j: (0, 0))
    return pl.pallas_call(
        _layer_body,
        grid=(B, NT, NT),
        in_specs=[
            pl.BlockSpec((1, TI, D), lambda b, i, j: (b, i, 0)),
            pl.BlockSpec((1, TJ, D), lambda b, i, j: (b, j, 0)),
            pl.BlockSpec((1, 1, TI, 8), lambda b, i, j: (b, i, 0, 0)),
            pl.BlockSpec((1, 1, TJ, 8), lambda b, i, j: (b, j, 0, 0)),
            wspec, bspec,
            pl.BlockSpec((RBF_PAD, D), lambda b, i, j: (0, 0)), bspec,
            wspec, bspec,
            wspec, bspec,
            wspec, bspec,
        ],
        out_specs=pl.BlockSpec((1, TI, D), lambda b, i, j: (b, i, 0)),
        out_shape=jax.ShapeDtypeStruct((B, N, D), jnp.float32),
        scratch_shapes=[pltpu.VMEM((TI, D), jnp.float32)],
    )(x, x, rp, rp,
      p['w_in'], p['b_in'].reshape(1, D),
      wf1p, p['b_f1'].reshape(1, D),
      p['w_f2'], p['b_f2'].reshape(1, D),
      p['w_o1'], p['b_o1'].reshape(1, D),
      p['w_o2'], p['b_o2'].reshape(1, D))


def kernel(Z, R, emb, params):
    x = emb[Z].astype(jnp.float32)
    rp = jnp.zeros((B, N, 8), jnp.float32).at[:, :, :3].set(R)
    rp = rp.reshape(B, NT, TI, 8)
    for p in params:
        wf1p = jnp.zeros((RBF_PAD, D), jnp.float32).at[:N_RBF].set(p['w_f1'])
        x = _interaction_layer(x, rp, p, wf1p)
    return x


# speedup vs baseline: 9.5371x; 1.0749x over previous
import math

import jax
import jax.numpy as jnp
from jax.experimental import pallas as pl
from jax.experimental.pallas import tpu as pltpu

B = 8
N = 256
D = 128
N_RBF = 20
RBF_PAD = 32
CUTOFF = 5.0
TI = 128
TJ = 128
NT = N // TI
LN2 = math.log(2.0)

_width = CUTOFF / (N_RBF - 1)
_COEFF = -0.5 / (_width * _width)
_SCALE = math.sqrt(-_COEFF)


def _ssp(v):
    return jax.nn.softplus(v) - LN2


def _layer_body(xi_ref, xj_ref, ri_ref, rj_ref, offs_ref, w_in_ref, b_in_ref,
                w_f1_ref, b_f1_ref, w_f2_ref, b_f2_ref,
                w_o1_ref, b_o1_ref, w_o2_ref, b_o2_ref,
                y_ref, acc_ref):
    i = pl.program_id(1)
    j = pl.program_id(2)

    @pl.when(j >= i)
    def _compute():
        ri = ri_ref[0, 0]
        rj = rj_ref[0, 0]
        dims = (((1,), (1,)), ((), ()))
        prod = jax.lax.dot_general(ri, rj, dims,
                                   preferred_element_type=jnp.float32)
        ri2 = jnp.sum(ri * ri, axis=1, keepdims=True)
        rj2 = jax.lax.dot_general(jnp.ones((1, 8), jnp.float32), rj * rj, dims,
                                  preferred_element_type=jnp.float32)
        d2 = ri2 + rj2 - 2.0 * prod
        d = jnp.sqrt(jnp.maximum(d2, 0.0) + 1e-12)

        ii = i * TI + jax.lax.broadcasted_iota(jnp.int32, (TI, TJ), 0)
        jj = j * TJ + jax.lax.broadcasted_iota(jnp.int32, (TI, TJ), 1)
        mask = (jj > ii) & (d < CUTOFF)
        rcut = jnp.where(mask, 0.5 * (jnp.cos(d * (math.pi / CUTOFF)) + 1.0), 0.0)

        ds = d * _SCALE
        delta = ds[:, :, None] - offs_ref[...]
        f = jnp.exp(-(delta * delta)).reshape(TI * TJ, RBF_PAD)

        t1 = jnp.dot(f, w_f1_ref[...], preferred_element_type=jnp.float32)
        t1 = _ssp(t1 + b_f1_ref[...])
        w = jnp.dot(t1, w_f2_ref[...], preferred_element_type=jnp.float32)
        w3 = (w + b_f2_ref[...]).reshape(TI, TJ, D) * rcut[:, :, None]

        h_j = jnp.dot(xj_ref[0], w_in_ref[...],
                      preferred_element_type=jnp.float32) + b_in_ref[...]
        msg = jnp.sum(w3 * h_j[None, :, :], axis=1)

        @pl.when(j == i)
        def _init():
            acc_ref[...] = msg

        @pl.when(j > i)
        def _add():
            acc_ref[...] = acc_ref[...] + msg

    @pl.when(j == NT - 1)
    def _finalize():
        agg = acc_ref[...]
        o = _ssp(jnp.dot(agg, w_o1_ref[...], preferred_element_type=jnp.float32)
                 + b_o1_ref[...])
        out = jnp.dot(o, w_o2_ref[...], preferred_element_type=jnp.float32) \
            + b_o2_ref[...]
        y_ref[0, :, :] = xi_ref[0] + out


def _interaction_layer(x, rp, p, wf1p, offs):
    wspec = pl.BlockSpec((D, D), lambda b, i, j: (0, 0))
    bspec = pl.BlockSpec((1, D), lambda b, i, j: (0, 0))
    return pl.pallas_call(
        _layer_body,
        grid=(B, NT, NT),
        in_specs=[
            pl.BlockSpec((1, TI, D), lambda b, i, j: (b, i, 0)),
            pl.BlockSpec((1, TJ, D), lambda b, i, j: (b, j, 0)),
            pl.BlockSpec((1, 1, TI, 8), lambda b, i, j: (b, i, 0, 0)),
            pl.BlockSpec((1, 1, TJ, 8), lambda b, i, j: (b, j, 0, 0)),
            pl.BlockSpec((1, 1, RBF_PAD), lambda b, i, j: (0, 0, 0)),
            wspec, bspec,
            pl.BlockSpec((RBF_PAD, D), lambda b, i, j: (0, 0)), bspec,
            wspec, bspec,
            wspec, bspec,
            wspec, bspec,
        ],
        out_specs=pl.BlockSpec((1, TI, D), lambda b, i, j: (b, i, 0)),
        out_shape=jax.ShapeDtypeStruct((B, N, D), jnp.float32),
        scratch_shapes=[pltpu.VMEM((TI, D), jnp.float32)],
    )(x, x, rp, rp, offs,
      p['w_in'], p['b_in'].reshape(1, D),
      wf1p, p['b_f1'].reshape(1, D),
      p['w_f2'], p['b_f2'].reshape(1, D),
      p['w_o1'], p['b_o1'].reshape(1, D),
      p['w_o2'], p['b_o2'].reshape(1, D))


def kernel(Z, R, emb, params):
    x = emb[Z].astype(jnp.float32)
    rp = jnp.zeros((B, N, 8), jnp.float32).at[:, :, :3].set(R)
    rp = rp.reshape(B, NT, TI, 8)
    ar = jnp.arange(RBF_PAD)
    offs = jnp.where(ar < N_RBF, ar * (_width * _SCALE), 1e6).astype(
        jnp.float32).reshape(1, 1, RBF_PAD)
    for p in params:
        wf1p = jnp.zeros((RBF_PAD, D), jnp.float32).at[:N_RBF].set(p['w_f1'])
        x = _interaction_layer(x, rp, p, wf1p, offs)
    return x


# speedup vs baseline: 10.5004x; 1.1010x over previous
import math

import jax
import jax.numpy as jnp
from jax.experimental import pallas as pl
from jax.experimental.pallas import tpu as pltpu

B = 8
N = 256
D = 128
N_RBF = 20
RBF_PAD = 32
CUTOFF = 5.0
TI = 128
TJ = 128
NT = N // TI
LN2 = math.log(2.0)

_width = CUTOFF / (N_RBF - 1)
_COEFF = -0.5 / (_width * _width)
_SCALE = math.sqrt(-_COEFF)


def _ssp(v):
    return jax.nn.softplus(v) - LN2


def _layer_body(xi_ref, xj_ref, ri_ref, rj_ref, offs_ref, w_in_ref, b_in_ref,
                w_f1_ref, b_f1_ref, w_f2_ref, b_f2_ref,
                w_o1_ref, b_o1_ref, w_o2_ref, b_o2_ref,
                y_ref, acc_ref):
    i = pl.program_id(1)
    j = pl.program_id(2)

    @pl.when(j >= i)
    def _compute():
        ri = ri_ref[0, 0]
        rj = rj_ref[0, 0]
        dims = (((1,), (1,)), ((), ()))
        prod = jax.lax.dot_general(ri, rj, dims,
                                   preferred_element_type=jnp.float32)
        ri2 = jnp.sum(ri * ri, axis=1, keepdims=True)
        rj2 = jax.lax.dot_general(jnp.ones((1, 8), jnp.float32), rj * rj, dims,
                                  preferred_element_type=jnp.float32)
        d2 = ri2 + rj2 - 2.0 * prod
        d = jnp.sqrt(jnp.maximum(d2, 0.0) + 1e-12)

        ii = i * TI + jax.lax.broadcasted_iota(jnp.int32, (TI, TJ), 0)
        jj = j * TJ + jax.lax.broadcasted_iota(jnp.int32, (TI, TJ), 1)
        mask = (jj > ii) & (d < CUTOFF)
        rcut = jnp.where(mask, 0.5 * (jnp.cos(d * (math.pi / CUTOFF)) + 1.0), 0.0)

        ds3 = (d * _SCALE).reshape(TI, 1, TJ)
        delta = ds3 - offs_ref[...]
        f3 = jnp.exp(-(delta * delta))
        t13 = jax.lax.dot_general(f3, w_f1_ref[...], (((1,), (0,)), ((), ())),
                                  preferred_element_type=jnp.float32)
        t1 = _ssp(t13.reshape(TI * TJ, D) + b_f1_ref[...])
        w = jnp.dot(t1, w_f2_ref[...], preferred_element_type=jnp.float32)
        w3 = (w + b_f2_ref[...]).reshape(TI, TJ, D) * rcut[:, :, None]

        h_j = jnp.dot(xj_ref[0], w_in_ref[...],
                      preferred_element_type=jnp.float32) + b_in_ref[...]
        msg = jnp.sum(w3 * h_j[None, :, :], axis=1)

        @pl.when(j == i)
        def _init():
            acc_ref[...] = msg

        @pl.when(j > i)
        def _add():
            acc_ref[...] = acc_ref[...] + msg

    @pl.when(j == NT - 1)
    def _finalize():
        agg = acc_ref[...]
        o = _ssp(jnp.dot(agg, w_o1_ref[...], preferred_element_type=jnp.float32)
                 + b_o1_ref[...])
        out = jnp.dot(o, w_o2_ref[...], preferred_element_type=jnp.float32) \
            + b_o2_ref[...]
        y_ref[0, :, :] = xi_ref[0] + out


def _interaction_layer(x, rp, p, wf1p, offs):
    wspec = pl.BlockSpec((D, D), lambda b, i, j: (0, 0))
    bspec = pl.BlockSpec((1, D), lambda b, i, j: (0, 0))
    return pl.pallas_call(
        _layer_body,
        grid=(B, NT, NT),
        in_specs=[
            pl.BlockSpec((1, TI, D), lambda b, i, j: (b, i, 0)),
            pl.BlockSpec((1, TJ, D), lambda b, i, j: (b, j, 0)),
            pl.BlockSpec((1, 1, TI, 8), lambda b, i, j: (b, i, 0, 0)),
            pl.BlockSpec((1, 1, TJ, 8), lambda b, i, j: (b, j, 0, 0)),
            pl.BlockSpec((1, RBF_PAD, 1), lambda b, i, j: (0, 0, 0)),
            wspec, bspec,
            pl.BlockSpec((RBF_PAD, D), lambda b, i, j: (0, 0)), bspec,
            wspec, bspec,
            wspec, bspec,
            wspec, bspec,
        ],
        out_specs=pl.BlockSpec((1, TI, D), lambda b, i, j: (b, i, 0)),
        out_shape=jax.ShapeDtypeStruct((B, N, D), jnp.float32),
        scratch_shapes=[pltpu.VMEM((TI, D), jnp.float32)],
    )(x, x, rp, rp, offs,
      p['w_in'], p['b_in'].reshape(1, D),
      wf1p, p['b_f1'].reshape(1, D),
      p['w_f2'], p['b_f2'].reshape(1, D),
      p['w_o1'], p['b_o1'].reshape(1, D),
      p['w_o2'], p['b_o2'].reshape(1, D))


def kernel(Z, R, emb, params):
    x = emb[Z].astype(jnp.float32)
    rp = jnp.zeros((B, N, 8), jnp.float32).at[:, :, :3].set(R)
    rp = rp.reshape(B, NT, TI, 8)
    ar = jnp.arange(RBF_PAD)
    offs = jnp.where(ar < N_RBF, ar * (_width * _SCALE), 1e6).astype(
        jnp.float32).reshape(1, RBF_PAD, 1)
    for p in params:
        wf1p = jnp.zeros((RBF_PAD, D), jnp.float32).at[:N_RBF].set(p['w_f1'])
        x = _interaction_layer(x, rp, p, wf1p, offs)
    return x


# speedup vs baseline: 11.3634x; 1.0822x over previous
import math

import jax
import jax.numpy as jnp
from jax.experimental import pallas as pl
from jax.experimental.pallas import tpu as pltpu

B = 8
N = 256
D = 128
N_RBF = 20
RBF_PAD = 24
CUTOFF = 5.0
TI = 128
TJ = 128
NT = N // TI
LN2 = math.log(2.0)

_width = CUTOFF / (N_RBF - 1)
_COEFF = -0.5 / (_width * _width)
_SCALE = math.sqrt(-_COEFF)


def _ssp(v):
    return jax.nn.softplus(v) - LN2


def _layer_body(xi_ref, xj_ref, ri_ref, rj_ref, offs_ref, w_in_ref, b_in_ref,
                w_f1_ref, b_f1_ref, w_f2_ref, b_f2_ref,
                w_o1_ref, b_o1_ref, w_o2_ref, b_o2_ref,
                y_ref, acc_ref):
    i = pl.program_id(1)
    j = pl.program_id(2)

    @pl.when(j >= i)
    def _compute():
        ri = ri_ref[0, 0]
        rj = rj_ref[0, 0]
        dims = (((1,), (1,)), ((), ()))
        prod = jax.lax.dot_general(ri, rj, dims,
                                   preferred_element_type=jnp.float32)
        ri2 = jnp.sum(ri * ri, axis=1, keepdims=True)
        rj2 = jax.lax.dot_general(jnp.ones((1, 8), jnp.float32), rj * rj, dims,
                                  preferred_element_type=jnp.float32)
        d2 = ri2 + rj2 - 2.0 * prod
        d = jnp.sqrt(jnp.maximum(d2, 0.0) + 1e-12)

        ii = i * TI + jax.lax.broadcasted_iota(jnp.int32, (TI, TJ), 0)
        jj = j * TJ + jax.lax.broadcasted_iota(jnp.int32, (TI, TJ), 1)
        mask = (jj > ii) & (d < CUTOFF)
        rcut = jnp.where(mask, 0.5 * (jnp.cos(d * (math.pi / CUTOFF)) + 1.0), 0.0)

        ds3 = (d * _SCALE).reshape(TI, 1, TJ)
        delta = ds3 - offs_ref[...]
        f3 = jnp.exp(-(delta * delta))
        t13 = jax.lax.dot_general(f3, w_f1_ref[...], (((1,), (0,)), ((), ())),
                                  preferred_element_type=jnp.float32)
        t1 = _ssp(t13.reshape(TI * TJ, D) + b_f1_ref[...])
        w = jnp.dot(t1, w_f2_ref[...], preferred_element_type=jnp.float32)

        h_j = jnp.dot(xj_ref[0], w_in_ref[...],
                      preferred_element_type=jnp.float32) + b_in_ref[...]
        u = (w + b_f2_ref[...]).reshape(TI, TJ, D) * h_j[None, :, :]
        msg = jax.lax.dot_general(rcut, u, (((1,), (1,)), ((0,), (0,))),
                                  preferred_element_type=jnp.float32)

        @pl.when(j == i)
        def _init():
            acc_ref[...] = msg

        @pl.when(j > i)
        def _add():
            acc_ref[...] = acc_ref[...] + msg

    @pl.when(j == NT - 1)
    def _finalize():
        agg = acc_ref[...]
        o = _ssp(jnp.dot(agg, w_o1_ref[...], preferred_element_type=jnp.float32)
                 + b_o1_ref[...])
        out = jnp.dot(o, w_o2_ref[...], preferred_element_type=jnp.float32) \
            + b_o2_ref[...]
        y_ref[0, :, :] = xi_ref[0] + out


def _interaction_layer(x, rp, p, wf1p, offs):
    wspec = pl.BlockSpec((D, D), lambda b, i, j: (0, 0))
    bspec = pl.BlockSpec((1, D), lambda b, i, j: (0, 0))
    return pl.pallas_call(
        _layer_body,
        grid=(B, NT, NT),
        in_specs=[
            pl.BlockSpec((1, TI, D), lambda b, i, j: (b, i, 0)),
            pl.BlockSpec((1, TJ, D), lambda b, i, j: (b, j, 0)),
            pl.BlockSpec((1, 1, TI, 8), lambda b, i, j: (b, i, 0, 0)),
            pl.BlockSpec((1, 1, TJ, 8), lambda b, i, j: (b, j, 0, 0)),
            pl.BlockSpec((1, RBF_PAD, 1), lambda b, i, j: (0, 0, 0)),
            wspec, bspec,
            pl.BlockSpec((RBF_PAD, D), lambda b, i, j: (0, 0)), bspec,
            wspec, bspec,
            wspec, bspec,
            wspec, bspec,
        ],
        out_specs=pl.BlockSpec((1, TI, D), lambda b, i, j: (b, i, 0)),
        out_shape=jax.ShapeDtypeStruct((B, N, D), jnp.float32),
        scratch_shapes=[pltpu.VMEM((TI, D), jnp.float32)],
    )(x, x, rp, rp, offs,
      p['w_in'], p['b_in'].reshape(1, D),
      wf1p, p['b_f1'].reshape(1, D),
      p['w_f2'], p['b_f2'].reshape(1, D),
      p['w_o1'], p['b_o1'].reshape(1, D),
      p['w_o2'], p['b_o2'].reshape(1, D))


def kernel(Z, R, emb, params):
    x = emb[Z].astype(jnp.float32)
    rp = jnp.zeros((B, N, 8), jnp.float32).at[:, :, :3].set(R)
    rp = rp.reshape(B, NT, TI, 8)
    ar = jnp.arange(RBF_PAD)
    offs = jnp.where(ar < N_RBF, ar * (_width * _SCALE), 1e6).astype(
        jnp.float32).reshape(1, RBF_PAD, 1)
    for p in params:
        wf1p = jnp.zeros((RBF_PAD, D), jnp.float32).at[:N_RBF].set(p['w_f1'])
        x = _interaction_layer(x, rp, p, wf1p, offs)
    return x


# speedup vs baseline: 15.8623x; 1.3959x over previous
import math

import jax
import jax.numpy as jnp
from jax.experimental import pallas as pl
from jax.experimental.pallas import tpu as pltpu

B = 8
N = 256
D = 128
N_RBF = 20
RBF_PAD = 24
CUTOFF = 5.0
TI = 128
TJ = 128
NT = N // TI
LN2 = math.log(2.0)

_width = CUTOFF / (N_RBF - 1)
_COEFF = -0.5 / (_width * _width)
_SCALE = math.sqrt(-_COEFF)


def _ssp(v):
    vc = jnp.minimum(v, 40.0)
    return jnp.log(1.0 + jnp.exp(vc)) - LN2


def _layer_body(xi_ref, xj_ref, ri_ref, rj_ref, offs_ref, w_in_ref, b_in_ref,
                w_f1_ref, b_f1_ref, w_f2_ref, b_f2_ref,
                w_o1_ref, b_o1_ref, w_o2_ref, b_o2_ref,
                y_ref, acc_ref):
    i = pl.program_id(1)
    j = pl.program_id(2)

    @pl.when(j >= i)
    def _compute():
        ri = ri_ref[0, 0]
        rj = rj_ref[0, 0]
        dims = (((1,), (1,)), ((), ()))
        prod = jax.lax.dot_general(ri, rj, dims,
                                   preferred_element_type=jnp.float32)
        ri2 = jnp.sum(ri * ri, axis=1, keepdims=True)
        rj2 = jax.lax.dot_general(jnp.ones((1, 8), jnp.float32), rj * rj, dims,
                                  preferred_element_type=jnp.float32)
        d2 = ri2 + rj2 - 2.0 * prod
        d = jnp.sqrt(jnp.maximum(d2, 0.0) + 1e-12)

        ii = i * TI + jax.lax.broadcasted_iota(jnp.int32, (TI, TJ), 0)
        jj = j * TJ + jax.lax.broadcasted_iota(jnp.int32, (TI, TJ), 1)
        mask = (jj > ii) & (d < CUTOFF)
        rcut = jnp.where(mask, 0.5 * (jnp.cos(d * (math.pi / CUTOFF)) + 1.0), 0.0)

        ds3 = (d * _SCALE).reshape(TI, 1, TJ)
        delta = ds3 - offs_ref[...]
        f3 = jnp.exp(-(delta * delta))
        w1b = jnp.broadcast_to(w_f1_ref[...][None], (TI, RBF_PAD, D))
        t13 = jax.lax.dot_general(f3, w1b, (((1,), (1,)), ((0,), (0,))),
                                  preferred_element_type=jnp.float32)
        t1 = _ssp(t13.reshape(TI * TJ, D) + b_f1_ref[...])
        w = jnp.dot(t1, w_f2_ref[...], preferred_element_type=jnp.float32)

        h_j = jnp.dot(xj_ref[0], w_in_ref[...],
                      preferred_element_type=jnp.float32) + b_in_ref[...]
        u = (w + b_f2_ref[...]).reshape(TI, TJ, D) * h_j[None, :, :]
        msg = jax.lax.dot_general(rcut, u, (((1,), (1,)), ((0,), (0,))),
                                  preferred_element_type=jnp.float32)

        @pl.when(j == i)
        def _init():
            acc_ref[...] = msg

        @pl.when(j > i)
        def _add():
            acc_ref[...] = acc_ref[...] + msg

    @pl.when(j == NT - 1)
    def _finalize():
        agg = acc_ref[...]
        o = _ssp(jnp.dot(agg, w_o1_ref[...], preferred_element_type=jnp.float32)
                 + b_o1_ref[...])
        out = jnp.dot(o, w_o2_ref[...], preferred_element_type=jnp.float32) \
            + b_o2_ref[...]
        y_ref[0, :, :] = xi_ref[0] + out


def _interaction_layer(x, rp, p, wf1p, offs):
    wspec = pl.BlockSpec((D, D), lambda b, i, j: (0, 0))
    bspec = pl.BlockSpec((1, D), lambda b, i, j: (0, 0))
    return pl.pallas_call(
        _layer_body,
        grid=(B, NT, NT),
        in_specs=[
            pl.BlockSpec((1, TI, D), lambda b, i, j: (b, i, 0)),
            pl.BlockSpec((1, TJ, D), lambda b, i, j: (b, j, 0)),
            pl.BlockSpec((1, 1, TI, 8), lambda b, i, j: (b, i, 0, 0)),
            pl.BlockSpec((1, 1, TJ, 8), lambda b, i, j: (b, j, 0, 0)),
            pl.BlockSpec((1, RBF_PAD, 1), lambda b, i, j: (0, 0, 0)),
            wspec, bspec,
            pl.BlockSpec((RBF_PAD, D), lambda b, i, j: (0, 0)), bspec,
            wspec, bspec,
            wspec, bspec,
            wspec, bspec,
        ],
        out_specs=pl.BlockSpec((1, TI, D), lambda b, i, j: (b, i, 0)),
        out_shape=jax.ShapeDtypeStruct((B, N, D), jnp.float32),
        scratch_shapes=[pltpu.VMEM((TI, D), jnp.float32)],
    )(x, x, rp, rp, offs,
      p['w_in'], p['b_in'].reshape(1, D),
      wf1p, p['b_f1'].reshape(1, D),
      p['w_f2'], p['b_f2'].reshape(1, D),
      p['w_o1'], p['b_o1'].reshape(1, D),
      p['w_o2'], p['b_o2'].reshape(1, D))


def kernel(Z, R, emb, params):
    x = emb[Z].astype(jnp.float32)
    rp = jnp.zeros((B, N, 8), jnp.float32).at[:, :, :3].set(R)
    rp = rp.reshape(B, NT, TI, 8)
    ar = jnp.arange(RBF_PAD)
    offs = jnp.where(ar < N_RBF, ar * (_width * _SCALE), 1e6).astype(
        jnp.float32).reshape(1, RBF_PAD, 1)
    for p in params:
        wf1p = jnp.zeros((RBF_PAD, D), jnp.float32).at[:N_RBF].set(p['w_f1'])
        x = _interaction_layer(x, rp, p, wf1p, offs)
    return x


# speedup vs baseline: 22.1942x; 1.3992x over previous
import math

import jax
import jax.numpy as jnp
from jax.experimental import pallas as pl
from jax.experimental.pallas import tpu as pltpu

B = 8
N = 256
D = 128
N_RBF = 20
RBF_PAD = 24
CUTOFF = 5.0
T = 128
LN2 = math.log(2.0)

_width = CUTOFF / (N_RBF - 1)
_COEFF = -0.5 / (_width * _width)
_SCALE = math.sqrt(-_COEFF)


def _ssp(v):
    vc = jnp.minimum(v, 40.0)
    return jnp.log(1.0 + jnp.exp(vc)) - LN2


def _dist(ra, rb):
    dims = (((1,), (1,)), ((), ()))
    prod = jax.lax.dot_general(ra, rb, dims, preferred_element_type=jnp.float32)
    ra2 = jnp.sum(ra * ra, axis=1, keepdims=True)
    rb2 = jax.lax.dot_general(jnp.ones((1, 8), jnp.float32), rb * rb, dims,
                              preferred_element_type=jnp.float32)
    d2 = ra2 + rb2 - 2.0 * prod
    return jnp.sqrt(jnp.maximum(d2, 0.0) + 1e-12)


def _rcut(d):
    return jnp.where(d < CUTOFF,
                     0.5 * (jnp.cos(d * (math.pi / CUTOFF)) + 1.0), 0.0)


def _filter_w3(d, offs_ref, w_f1_ref, b_f1_ref, w_f2_ref, b_f2_ref):
    ds3 = (d * _SCALE).reshape(T, 1, T)
    delta = ds3 - offs_ref[...]
    f3 = jnp.exp(-(delta * delta))
    w1b = jnp.broadcast_to(w_f1_ref[...][None], (T, RBF_PAD, D))
    t13 = jax.lax.dot_general(f3, w1b, (((1,), (1,)), ((0,), (0,))),
                              preferred_element_type=jnp.float32)
    t1 = _ssp(t13.reshape(T * T, D) + b_f1_ref[...])
    w = jnp.dot(t1, w_f2_ref[...], preferred_element_type=jnp.float32)
    return (w + b_f2_ref[...]).reshape(T, T, D)


def _layer_body(x_ref, rp_ref, offs_ref, w_in_ref, b_in_ref,
                w_f1_ref, b_f1_ref, w_f2_ref, b_f2_ref,
                w_o1_ref, b_o1_ref, w_o2_ref, b_o2_ref,
                y_ref, acc_ref, h_ref):
    t = pl.program_id(1)
    fargs = (offs_ref, w_f1_ref, b_f1_ref, w_f2_ref, b_f2_ref)

    @pl.when(t == 0)
    def _packed_diag():
        h_ref[...] = jnp.dot(x_ref[0], w_in_ref[...],
                             preferred_element_type=jnp.float32) + b_in_ref[...]
        r0 = rp_ref[0, 0]
        r1 = rp_ref[0, 1]
        d00 = _dist(r0, r0)
        d11 = _dist(r1, r1)
        pp = jax.lax.broadcasted_iota(jnp.int32, (T, T), 0)
        qq = jax.lax.broadcasted_iota(jnp.int32, (T, T), 1)
        d = jnp.where(qq > pp, d00, d11)
        rc = _rcut(d)
        rc_u = jnp.where(qq > pp, rc, 0.0)
        rc_l = jnp.where(qq < pp, rc, 0.0)
        w3 = _filter_w3(d, *fargs)
        h0 = h_ref[0:T, :]
        h1 = h_ref[T:N, :]
        msg_u = jax.lax.dot_general(rc_u, w3 * h0[None, :, :],
                                    (((1,), (1,)), ((0,), (0,))),
                                    preferred_element_type=jnp.float32)
        msg_l = jax.lax.dot_general(rc_l, w3 * h1[:, None, :],
                                    (((0,), (0,)), ((1,), (1,))),
                                    preferred_element_type=jnp.float32)
        acc_ref[0:T, :] = msg_u
        acc_ref[T:N, :] = msg_l

    @pl.when(t == 1)
    def _offdiag():
        r0 = rp_ref[0, 0]
        r1 = rp_ref[0, 1]
        d = _dist(r0, r1)
        rc = _rcut(d)
        w3 = _filter_w3(d, *fargs)
        h1 = h_ref[T:N, :]
        msg = jax.lax.dot_general(rc, w3 * h1[None, :, :],
                                  (((1,), (1,)), ((0,), (0,))),
                                  preferred_element_type=jnp.float32)
        acc_ref[0:T, :] = acc_ref[0:T, :] + msg
        agg = acc_ref[...]
        o = _ssp(jnp.dot(agg, w_o1_ref[...], preferred_element_type=jnp.float32)
                 + b_o1_ref[...])
        out = jnp.dot(o, w_o2_ref[...], preferred_element_type=jnp.float32) \
            + b_o2_ref[...]
        y_ref[0, :, :] = x_ref[0] + out


def _interaction_layer(x, rp, p, wf1p, offs):
    wspec = pl.BlockSpec((D, D), lambda b, t: (0, 0))
    bspec = pl.BlockSpec((1, D), lambda b, t: (0, 0))
    return pl.pallas_call(
        _layer_body,
        grid=(B, 2),
        in_specs=[
            pl.BlockSpec((1, N, D), lambda b, t: (b, 0, 0)),
            pl.BlockSpec((1, 2, T, 8), lambda b, t: (b, 0, 0, 0)),
            pl.BlockSpec((1, RBF_PAD, 1), lambda b, t: (0, 0, 0)),
            wspec, bspec,
            pl.BlockSpec((RBF_PAD, D), lambda b, t: (0, 0)), bspec,
            wspec, bspec,
            wspec, bspec,
            wspec, bspec,
        ],
        out_specs=pl.BlockSpec((1, N, D), lambda b, t: (b, 0, 0)),
        out_shape=jax.ShapeDtypeStruct((B, N, D), jnp.float32),
        scratch_shapes=[pltpu.VMEM((N, D), jnp.float32),
                        pltpu.VMEM((N, D), jnp.float32)],
    )(x, rp, offs,
      p['w_in'], p['b_in'].reshape(1, D),
      wf1p, p['b_f1'].reshape(1, D),
      p['w_f2'], p['b_f2'].reshape(1, D),
      p['w_o1'], p['b_o1'].reshape(1, D),
      p['w_o2'], p['b_o2'].reshape(1, D))


def kernel(Z, R, emb, params):
    x = emb[Z].astype(jnp.float32)
    rp = jnp.zeros((B, N, 8), jnp.float32).at[:, :, :3].set(R)
    rp = rp.reshape(B, 2, T, 8)
    ar = jnp.arange(RBF_PAD)
    offs = jnp.where(ar < N_RBF, ar * (_width * _SCALE), 1e6).astype(
        jnp.float32).reshape(1, RBF_PAD, 1)
    for p in params:
        wf1p = jnp.zeros((RBF_PAD, D), jnp.float32).at[:N_RBF].set(p['w_f1'])
        x = _interaction_layer(x, rp, p, wf1p, offs)
    return x


# speedup vs baseline: 25.2773x; 1.1389x over previous
import math

import jax
import jax.numpy as jnp
from jax.experimental import pallas as pl
from jax.experimental.pallas import tpu as pltpu

B = 8
N = 256
D = 128
N_RBF = 20
RBF_PAD = 24
CUTOFF = 5.0
T = 128
LN2 = math.log(2.0)

_width = CUTOFF / (N_RBF - 1)
_COEFF = -0.5 / (_width * _width)
_SCALE = math.sqrt(-_COEFF)


def _ssp(v):
    vc = jnp.minimum(v, 40.0)
    return jnp.log(1.0 + jnp.exp(vc)) - LN2


def _dist(ra, rb):
    dims = (((1,), (1,)), ((), ()))
    prod = jax.lax.dot_general(ra, rb, dims, preferred_element_type=jnp.float32)
    ra2 = jnp.sum(ra * ra, axis=1, keepdims=True)
    rb2 = jax.lax.dot_general(jnp.ones((1, 8), jnp.float32), rb * rb, dims,
                              preferred_element_type=jnp.float32)
    d2 = ra2 + rb2 - 2.0 * prod
    return jnp.sqrt(jnp.maximum(d2, 0.0) + 1e-12)


def _rcut(d):
    return jnp.where(d < CUTOFF,
                     0.5 * (jnp.cos(d * (math.pi / CUTOFF)) + 1.0), 0.0)


def _filter_w3(d, offs_ref, w_f1_ref, b_f1_ref, w_f2_ref, b_f2_ref):
    ds3 = (d * _SCALE).reshape(T, 1, T)
    delta = ds3 - offs_ref[...]
    f3 = jnp.exp(-(delta * delta))
    w1b = jnp.broadcast_to(w_f1_ref[...][None], (T, RBF_PAD, D))
    t13 = jax.lax.dot_general(f3, w1b, (((1,), (1,)), ((0,), (0,))),
                              preferred_element_type=jnp.float32)
    t1 = _ssp(t13.reshape(T * T, D) + b_f1_ref[...])
    w = jnp.dot(t1, w_f2_ref[...], preferred_element_type=jnp.float32)
    return (w + b_f2_ref[...]).reshape(T, T, D)


def _layer_body(x_ref, rp_ref, offs_ref, w_in_ref, b_in_ref,
                w_f1_ref, b_f1_ref, w_f2_ref, b_f2_ref,
                w_o1_ref, b_o1_ref, w_o2_ref, b_o2_ref,
                y_ref, acc_ref, h_ref):
    t = pl.program_id(1)
    fargs = (offs_ref, w_f1_ref, b_f1_ref, w_f2_ref, b_f2_ref)

    @pl.when(t == 0)
    def _packed_diag():
        h_ref[...] = jnp.dot(x_ref[0], w_in_ref[...],
                             preferred_element_type=jnp.float32) + b_in_ref[...]
        r0 = rp_ref[0, 0]
        r1 = rp_ref[0, 1]
        d00 = _dist(r0, r0)
        d11 = _dist(r1, r1)
        pp = jax.lax.broadcasted_iota(jnp.int32, (T, T), 0)
        qq = jax.lax.broadcasted_iota(jnp.int32, (T, T), 1)
        d = jnp.where(qq > pp, d00, d11)
        rc = _rcut(d)
        rc_u = jnp.where(qq > pp, rc, 0.0)
        rc_l = jnp.where(qq < pp, rc, 0.0)
        w3 = _filter_w3(d, *fargs)
        h0 = h_ref[0:T, :]
        h1 = h_ref[T:N, :]
        msg_u = jax.lax.dot_general(rc_u, w3 * h0[None, :, :],
                                    (((1,), (1,)), ((0,), (0,))),
                                    preferred_element_type=jnp.float32)
        msg_l = jnp.sum(w3 * (rc_l[:, :, None] * h1[:, None, :]), axis=0)
        acc_ref[0:T, :] = msg_u
        acc_ref[T:N, :] = msg_l

    @pl.when(t == 1)
    def _offdiag():
        r0 = rp_ref[0, 0]
        r1 = rp_ref[0, 1]
        d = _dist(r0, r1)
        rc = _rcut(d)
        w3 = _filter_w3(d, *fargs)
        h1 = h_ref[T:N, :]
        msg = jax.lax.dot_general(rc, w3 * h1[None, :, :],
                                  (((1,), (1,)), ((0,), (0,))),
                                  preferred_element_type=jnp.float32)
        acc_ref[0:T, :] = acc_ref[0:T, :] + msg
        agg = acc_ref[...]
        o = _ssp(jnp.dot(agg, w_o1_ref[...], preferred_element_type=jnp.float32)
                 + b_o1_ref[...])
        out = jnp.dot(o, w_o2_ref[...], preferred_element_type=jnp.float32) \
            + b_o2_ref[...]
        y_ref[0, :, :] = x_ref[0] + out


def _interaction_layer(x, rp, p, wf1p, offs):
    wspec = pl.BlockSpec((D, D), lambda b, t: (0, 0))
    bspec = pl.BlockSpec((1, D), lambda b, t: (0, 0))
    return pl.pallas_call(
        _layer_body,
        grid=(B, 2),
        in_specs=[
            pl.BlockSpec((1, N, D), lambda b, t: (b, 0, 0)),
            pl.BlockSpec((1, 2, T, 8), lambda b, t: (b, 0, 0, 0)),
            pl.BlockSpec((1, RBF_PAD, 1), lambda b, t: (0, 0, 0)),
            wspec, bspec,
            pl.BlockSpec((RBF_PAD, D), lambda b, t: (0, 0)), bspec,
            wspec, bspec,
            wspec, bspec,
            wspec, bspec,
        ],
        out_specs=pl.BlockSpec((1, N, D), lambda b, t: (b, 0, 0)),
        out_shape=jax.ShapeDtypeStruct((B, N, D), jnp.float32),
        scratch_shapes=[pltpu.VMEM((N, D), jnp.float32),
                        pltpu.VMEM((N, D), jnp.float32)],
    )(x, rp, offs,
      p['w_in'], p['b_in'].reshape(1, D),
      wf1p, p['b_f1'].reshape(1, D),
      p['w_f2'], p['b_f2'].reshape(1, D),
      p['w_o1'], p['b_o1'].reshape(1, D),
      p['w_o2'], p['b_o2'].reshape(1, D))


def kernel(Z, R, emb, params):
    x = emb[Z].astype(jnp.float32)
    rp = jnp.zeros((B, N, 8), jnp.float32).at[:, :, :3].set(R)
    rp = rp.reshape(B, 2, T, 8)
    ar = jnp.arange(RBF_PAD)
    offs = jnp.where(ar < N_RBF, ar * (_width * _SCALE), 1e6).astype(
        jnp.float32).reshape(1, RBF_PAD, 1)
    for p in params:
        wf1p = jnp.zeros((RBF_PAD, D), jnp.float32).at[:N_RBF].set(p['w_f1'])
        x = _interaction_layer(x, rp, p, wf1p, offs)
    return x
